# Initial kernel scaffold; baseline (speedup 1.0000x reference)
import jax
import jax.numpy as jnp
from jax.experimental import pallas as pl


def kernel(input, codebooks):
    raise NotImplementedError("write your pallas kernel here")


# speedup vs baseline: 1.2287x; 1.2287x over previous
import jax
import jax.numpy as jnp
from jax.experimental import pallas as pl
from jax.experimental.pallas import tpu as pltpu

_TB = 512


def _rvq_body(x_ref, cb_ref, cbt_ref, out_ref, idx_ref):
    r = x_ref[0]
    acc = jnp.zeros_like(r)
    S, K, _D = cb_ref.shape
    TB = r.shape[1]
    kiota = jax.lax.broadcasted_iota(jnp.int32, (K, TB), 0)
    for s in range(S):
        cb = cb_ref[s]
        e2 = jnp.sum(cb * cb, axis=1, keepdims=True)
        r2 = jnp.sum(r * r, axis=0, keepdims=True)
        dots = jnp.dot(cb.astype(jnp.bfloat16), r.astype(jnp.bfloat16),
                       preferred_element_type=jnp.float32)
        dist = (r2 - 2.0 * dots) + e2
        mind = jnp.min(dist, axis=0, keepdims=True)
        idx = jnp.min(jnp.where(dist == mind, kiota, K), axis=0, keepdims=True)
        onehot = (kiota == idx).astype(jnp.float32)
        q = jnp.dot(cbt_ref[s], onehot, preferred_element_type=jnp.float32,
                    precision=jax.lax.Precision.HIGHEST)
        acc = acc + q
        r = r - q
        idx_ref[0, pl.ds(s, 1), :] = idx
    out_ref[0] = acc


def kernel(input, codebooks):
    B, D, T = input.shape
    S, K, _ = codebooks.shape
    cbt = jnp.transpose(codebooks, (0, 2, 1))
    grid = (B, T // _TB)
    out, idx = pl.pallas_call(
        _rvq_body,
        grid=grid,
        in_specs=[
            pl.BlockSpec((1, D, _TB), lambda b, t: (b, 0, t)),
            pl.BlockSpec((S, K, D), lambda b, t: (0, 0, 0)),
            pl.BlockSpec((S, D, K), lambda b, t: (0, 0, 0)),
        ],
        out_specs=[
            pl.BlockSpec((1, D, _TB), lambda b, t: (b, 0, t)),
            pl.BlockSpec((1, S, _TB), lambda b, t: (b, 0, t)),
        ],
        out_shape=[
            jax.ShapeDtypeStruct((B, D, T), jnp.float32),
            jax.ShapeDtypeStruct((B, S, T), jnp.int32),
        ],
        compiler_params=pltpu.CompilerParams(
            dimension_semantics=("parallel", "parallel"),
        ),
    )(input, codebooks, cbt)
    return out, idx


# speedup vs baseline: 2.4959x; 2.0314x over previous
import jax
import jax.numpy as jnp
from jax.experimental import pallas as pl
from jax.experimental.pallas import tpu as pltpu

_TB = 512


def _rvq_body(x_ref, cbm2_ref, e2_ref, planes_ref, out_ref, idx_ref):
    r = x_ref[0]
    acc = jnp.zeros_like(r)
    S, K, D = cbm2_ref.shape
    TB = r.shape[1]
    kiota = jax.lax.broadcasted_iota(jnp.int32, (K, TB), 0)
    for s in range(S):
        r2 = jnp.sum(r * r, axis=0, keepdims=True)
        rb = r.astype(jnp.bfloat16)
        dots2 = jnp.dot(cbm2_ref[s], rb, preferred_element_type=jnp.float32)
        dist = (r2 + dots2) + e2_ref[s]
        mind = jnp.min(dist, axis=0, keepdims=True)
        masked = jnp.where(dist == mind, kiota, K)
        idx = jnp.min(masked, axis=0, keepdims=True)
        onehot = (kiota == idx).astype(jnp.bfloat16)
        q3 = jnp.dot(planes_ref[s], onehot, preferred_element_type=jnp.float32)
        q = (q3[:D, :] + q3[D:2 * D, :]) + q3[2 * D:, :]
        acc = acc + q
        r = r - q
        idx_ref[0, pl.ds(s, 1), :] = idx
    out_ref[0] = acc


def kernel(input, codebooks):
    B, D, T = input.shape
    S, K, _ = codebooks.shape
    cbm2 = (-2.0 * codebooks).astype(jnp.bfloat16)
    e2 = jnp.sum(codebooks ** 2, axis=-1, keepdims=True)

    def _trunc_split(v):
        b = jax.lax.bitcast_convert_type(v, jnp.uint32)
        plane_f = jax.lax.bitcast_convert_type(
            b & jnp.uint32(0xFFFF0000), jnp.float32)
        plane_b = jax.lax.bitcast_convert_type(
            (b >> 16).astype(jnp.uint16), jnp.bfloat16)
        return plane_b, v - plane_f

    hi, rem = _trunc_split(codebooks)
    mid, lo_f = _trunc_split(rem)
    lo, _ = _trunc_split(lo_f)
    planes = jnp.concatenate(
        [jnp.transpose(hi, (0, 2, 1)),
         jnp.transpose(mid, (0, 2, 1)),
         jnp.transpose(lo, (0, 2, 1))], axis=1)
    grid = (B, T // _TB)
    out, idx = pl.pallas_call(
        _rvq_body,
        grid=grid,
        in_specs=[
            pl.BlockSpec((1, D, _TB), lambda b, t: (b, 0, t)),
            pl.BlockSpec((S, K, D), lambda b, t: (0, 0, 0)),
            pl.BlockSpec((S, K, 1), lambda b, t: (0, 0, 0)),
            pl.BlockSpec((S, 3 * D, K), lambda b, t: (0, 0, 0)),
        ],
        out_specs=[
            pl.BlockSpec((1, D, _TB), lambda b, t: (b, 0, t)),
            pl.BlockSpec((1, S, _TB), lambda b, t: (b, 0, t)),
        ],
        out_shape=[
            jax.ShapeDtypeStruct((B, D, T), jnp.float32),
            jax.ShapeDtypeStruct((B, S, T), jnp.int32),
        ],
        compiler_params=pltpu.CompilerParams(
            dimension_semantics=("parallel", "parallel"),
        ),
    )(input, cbm2, e2, planes)
    return out, idx


# speedup vs baseline: 3.1879x; 1.2773x over previous
import jax
import jax.numpy as jnp
from jax.experimental import pallas as pl
from jax.experimental.pallas import tpu as pltpu

_TB = 512


def _rvq_body(x_ref, cbm2_ref, e2_ref, planes_ref, out_ref, idx_ref):
    r = x_ref[0]
    acc = jnp.zeros_like(r)
    S, K, D = cbm2_ref.shape
    TB = r.shape[1]
    kiota = jax.lax.broadcasted_iota(jnp.int32, (K, TB), 0).astype(jnp.float32)
    for s in range(S):
        r2 = jnp.sum(r * r, axis=0, keepdims=True)
        rb = r.astype(jnp.bfloat16)
        dots2 = jnp.dot(cbm2_ref[s], rb, preferred_element_type=jnp.float32)
        dist = (r2 + dots2) + e2_ref[s]
        idx = jnp.argmin(dist, axis=0).astype(jnp.float32)[None, :]
        onehot = (kiota == idx).astype(jnp.bfloat16)
        q3 = jnp.dot(planes_ref[s], onehot, preferred_element_type=jnp.float32)
        q = (q3[:D, :] + q3[D:2 * D, :]) + q3[2 * D:, :]
        acc = acc + q
        r = r - q
        idx_ref[0, pl.ds(s, 1), :] = idx.astype(jnp.int32)
    out_ref[0] = acc


def kernel(input, codebooks):
    B, D, T = input.shape
    S, K, _ = codebooks.shape
    cbm2 = (-2.0 * codebooks).astype(jnp.bfloat16)
    e2 = jnp.sum(codebooks ** 2, axis=-1, keepdims=True)

    def _trunc_split(v):
        b = jax.lax.bitcast_convert_type(v, jnp.uint32)
        plane_f = jax.lax.bitcast_convert_type(
            b & jnp.uint32(0xFFFF0000), jnp.float32)
        plane_b = jax.lax.bitcast_convert_type(
            (b >> 16).astype(jnp.uint16), jnp.bfloat16)
        return plane_b, v - plane_f

    hi, rem = _trunc_split(codebooks)
    mid, lo_f = _trunc_split(rem)
    lo, _ = _trunc_split(lo_f)
    planes = jnp.concatenate(
        [jnp.transpose(hi, (0, 2, 1)),
         jnp.transpose(mid, (0, 2, 1)),
         jnp.transpose(lo, (0, 2, 1))], axis=1)
    grid = (B, T // _TB)
    out, idx = pl.pallas_call(
        _rvq_body,
        grid=grid,
        in_specs=[
            pl.BlockSpec((1, D, _TB), lambda b, t: (b, 0, t)),
            pl.BlockSpec((S, K, D), lambda b, t: (0, 0, 0)),
            pl.BlockSpec((S, K, 1), lambda b, t: (0, 0, 0)),
            pl.BlockSpec((S, 3 * D, K), lambda b, t: (0, 0, 0)),
        ],
        out_specs=[
            pl.BlockSpec((1, D, _TB), lambda b, t: (b, 0, t)),
            pl.BlockSpec((1, S, _TB), lambda b, t: (b, 0, t)),
        ],
        out_shape=[
            jax.ShapeDtypeStruct((B, D, T), jnp.float32),
            jax.ShapeDtypeStruct((B, S, T), jnp.int32),
        ],
        compiler_params=pltpu.CompilerParams(
            dimension_semantics=("parallel", "parallel"),
        ),
    )(input, cbm2, e2, planes)
    return out, idx


# speedup vs baseline: 4.8692x; 1.5274x over previous
import jax
import jax.numpy as jnp
from jax.experimental import pallas as pl
from jax.experimental.pallas import tpu as pltpu

_TB = 2048


def _rvq_body(x_ref, cbm2_ref, e2_ref, planes_ref, out_ref, idx_ref):
    r = x_ref[0]
    acc = jnp.zeros_like(r)
    S, K, D = cbm2_ref.shape
    TB = r.shape[1]
    kiota = jax.lax.broadcasted_iota(jnp.int32, (K, TB), 0)
    for s in range(S):
        r2 = jnp.sum(r * r, axis=0, keepdims=True)
        rb = r.astype(jnp.bfloat16)
        dots2 = jnp.dot(cbm2_ref[s], rb, preferred_element_type=jnp.float32)
        dist = (r2 + dots2) + e2_ref[s]
        idx = jnp.argmin(dist, axis=0)[None, :]
        onehot = (kiota == idx).astype(jnp.bfloat16)
        q3 = jnp.dot(planes_ref[s], onehot, preferred_element_type=jnp.float32)
        q = (q3[:D, :] + q3[D:2 * D, :]) + q3[2 * D:, :]
        acc = acc + q
        r = r - q
        idx_ref[0, pl.ds(s, 1), :] = idx
    out_ref[0] = acc


def kernel(input, codebooks):
    B, D, T = input.shape
    S, K, _ = codebooks.shape
    cbm2 = (-2.0 * codebooks).astype(jnp.bfloat16)
    e2 = jnp.sum(codebooks ** 2, axis=-1, keepdims=True)

    def _trunc_split(v):
        b = jax.lax.bitcast_convert_type(v, jnp.uint32)
        plane_f = jax.lax.bitcast_convert_type(
            b & jnp.uint32(0xFFFF0000), jnp.float32)
        plane_b = jax.lax.bitcast_convert_type(
            (b >> 16).astype(jnp.uint16), jnp.bfloat16)
        return plane_b, v - plane_f

    hi, rem = _trunc_split(codebooks)
    mid, lo_f = _trunc_split(rem)
    lo, _ = _trunc_split(lo_f)
    planes = jnp.concatenate(
        [jnp.transpose(hi, (0, 2, 1)),
         jnp.transpose(mid, (0, 2, 1)),
         jnp.transpose(lo, (0, 2, 1))], axis=1)
    grid = (B, T // _TB)
    out, idx = pl.pallas_call(
        _rvq_body,
        grid=grid,
        in_specs=[
            pl.BlockSpec((1, D, _TB), lambda b, t: (b, 0, t)),
            pl.BlockSpec((S, K, D), lambda b, t: (0, 0, 0)),
            pl.BlockSpec((S, K, 1), lambda b, t: (0, 0, 0)),
            pl.BlockSpec((S, 3 * D, K), lambda b, t: (0, 0, 0)),
        ],
        out_specs=[
            pl.BlockSpec((1, D, _TB), lambda b, t: (b, 0, t)),
            pl.BlockSpec((1, S, _TB), lambda b, t: (b, 0, t)),
        ],
        out_shape=[
            jax.ShapeDtypeStruct((B, D, T), jnp.float32),
            jax.ShapeDtypeStruct((B, S, T), jnp.int32),
        ],
        compiler_params=pltpu.CompilerParams(
            dimension_semantics=("parallel", "parallel"),
        ),
    )(input, cbm2, e2, planes)
    return out, idx


# speedup vs baseline: 4.9410x; 1.0147x over previous
import jax
import jax.numpy as jnp
from jax.experimental import pallas as pl
from jax.experimental.pallas import tpu as pltpu

_TB = 2048


def _rvq_body(x_ref, cbm2_ref, e2_ref, planes_ref, out_ref, idx_ref):
    r = x_ref[0]
    acc = jnp.zeros_like(r)
    S, K, D = cbm2_ref.shape
    TB = r.shape[1]
    kiota = jax.lax.broadcasted_iota(jnp.int32, (K, TB), 0).astype(jnp.int16)
    for s in range(S):
        r2 = jnp.sum(r * r, axis=0, keepdims=True)
        rb = r.astype(jnp.bfloat16)
        dots2 = jnp.dot(cbm2_ref[s], rb, preferred_element_type=jnp.float32)
        dist = (r2 + dots2) + e2_ref[s]
        idx = jnp.argmin(dist, axis=0)[None, :]
        onehot = jnp.where(kiota == idx.astype(jnp.int16),
                           jnp.bfloat16(1), jnp.bfloat16(0))
        q3 = jnp.dot(planes_ref[s], onehot, preferred_element_type=jnp.float32)
        q = (q3[:D, :] + q3[D:2 * D, :]) + q3[2 * D:, :]
        acc = acc + q
        r = r - q
        idx_ref[0, pl.ds(s, 1), :] = idx
    out_ref[0] = acc


def kernel(input, codebooks):
    B, D, T = input.shape
    S, K, _ = codebooks.shape
    cbm2 = (-2.0 * codebooks).astype(jnp.bfloat16)
    e2 = jnp.sum(codebooks ** 2, axis=-1, keepdims=True)

    def _trunc_split(v):
        b = jax.lax.bitcast_convert_type(v, jnp.uint32)
        plane_f = jax.lax.bitcast_convert_type(
            b & jnp.uint32(0xFFFF0000), jnp.float32)
        plane_b = jax.lax.bitcast_convert_type(
            (b >> 16).astype(jnp.uint16), jnp.bfloat16)
        return plane_b, v - plane_f

    hi, rem = _trunc_split(codebooks)
    mid, lo_f = _trunc_split(rem)
    lo, _ = _trunc_split(lo_f)
    planes = jnp.concatenate(
        [jnp.transpose(hi, (0, 2, 1)),
         jnp.transpose(mid, (0, 2, 1)),
         jnp.transpose(lo, (0, 2, 1))], axis=1)
    grid = (B, T // _TB)
    out, idx = pl.pallas_call(
        _rvq_body,
        grid=grid,
        in_specs=[
            pl.BlockSpec((1, D, _TB), lambda b, t: (b, 0, t)),
            pl.BlockSpec((S, K, D), lambda b, t: (0, 0, 0)),
            pl.BlockSpec((S, K, 1), lambda b, t: (0, 0, 0)),
            pl.BlockSpec((S, 3 * D, K), lambda b, t: (0, 0, 0)),
        ],
        out_specs=[
            pl.BlockSpec((1, D, _TB), lambda b, t: (b, 0, t)),
            pl.BlockSpec((1, S, _TB), lambda b, t: (b, 0, t)),
        ],
        out_shape=[
            jax.ShapeDtypeStruct((B, D, T), jnp.float32),
            jax.ShapeDtypeStruct((B, S, T), jnp.int32),
        ],
        compiler_params=pltpu.CompilerParams(
            dimension_semantics=("parallel", "parallel"),
        ),
    )(input, cbm2, e2, planes)
    return out, idx


# speedup vs baseline: 4.9669x; 1.0052x over previous
import jax
import jax.numpy as jnp
from jax.experimental import pallas as pl
from jax.experimental.pallas import tpu as pltpu

_TB = 4096


def _rvq_body(x_ref, cbm2_ref, e2_ref, planes_ref, out_ref, idx_ref):
    r = x_ref[0]
    acc = jnp.zeros_like(r)
    S, K, D = cbm2_ref.shape
    TB = r.shape[1]
    kiota = jax.lax.broadcasted_iota(jnp.int32, (K, TB), 0).astype(jnp.int16)
    for s in range(S):
        r2 = jnp.sum(r * r, axis=0, keepdims=True)
        rb = r.astype(jnp.bfloat16)
        dots2 = jnp.dot(cbm2_ref[s], rb, preferred_element_type=jnp.float32)
        dist = (r2 + dots2) + e2_ref[s]
        idx = jnp.argmin(dist, axis=0)[None, :]
        onehot = jnp.where(kiota == idx.astype(jnp.int16),
                           jnp.bfloat16(1), jnp.bfloat16(0))
        q3 = jnp.dot(planes_ref[s], onehot, preferred_element_type=jnp.float32)
        q = (q3[:D, :] + q3[D:2 * D, :]) + q3[2 * D:, :]
        acc = acc + q
        r = r - q
        idx_ref[0, pl.ds(s, 1), :] = idx
    out_ref[0] = acc


def kernel(input, codebooks):
    B, D, T = input.shape
    S, K, _ = codebooks.shape
    cbm2 = (-2.0 * codebooks).astype(jnp.bfloat16)
    e2 = jnp.sum(codebooks ** 2, axis=-1, keepdims=True)

    def _trunc_split(v):
        b = jax.lax.bitcast_convert_type(v, jnp.uint32)
        plane_f = jax.lax.bitcast_convert_type(
            b & jnp.uint32(0xFFFF0000), jnp.float32)
        plane_b = jax.lax.bitcast_convert_type(
            (b >> 16).astype(jnp.uint16), jnp.bfloat16)
        return plane_b, v - plane_f

    hi, rem = _trunc_split(codebooks)
    mid, lo_f = _trunc_split(rem)
    lo, _ = _trunc_split(lo_f)
    planes = jnp.concatenate(
        [jnp.transpose(hi, (0, 2, 1)),
         jnp.transpose(mid, (0, 2, 1)),
         jnp.transpose(lo, (0, 2, 1))], axis=1)
    grid = (B, T // _TB)
    out, idx = pl.pallas_call(
        _rvq_body,
        grid=grid,
        in_specs=[
            pl.BlockSpec((1, D, _TB), lambda b, t: (b, 0, t)),
            pl.BlockSpec((S, K, D), lambda b, t: (0, 0, 0)),
            pl.BlockSpec((S, K, 1), lambda b, t: (0, 0, 0)),
            pl.BlockSpec((S, 3 * D, K), lambda b, t: (0, 0, 0)),
        ],
        out_specs=[
            pl.BlockSpec((1, D, _TB), lambda b, t: (b, 0, t)),
            pl.BlockSpec((1, S, _TB), lambda b, t: (b, 0, t)),
        ],
        out_shape=[
            jax.ShapeDtypeStruct((B, D, T), jnp.float32),
            jax.ShapeDtypeStruct((B, S, T), jnp.int32),
        ],
        compiler_params=pltpu.CompilerParams(
            dimension_semantics=("parallel", "parallel"),
        ),
    )(input, cbm2, e2, planes)
    return out, idx
